# Initial kernel scaffold; baseline (speedup 1.0000x reference)
import jax
import jax.numpy as jnp
from jax.experimental import pallas as pl


def kernel(x_in, table, W, b):
    raise NotImplementedError("write your pallas kernel here")


# speedup vs baseline: 13.5327x; 13.5327x over previous
import functools
import jax
import jax.numpy as jnp
from jax import lax
from jax.experimental import pallas as pl
from jax.experimental.pallas import tpu as pltpu
from jax.experimental.pallas import tpu_sc as plsc

VOCAB_N = 1000000
D_N = 32
B_N = 16384
L_N = 200

ROWS2 = VOCAB_N * D_N // 128
BLK1 = 2000
GRID1 = ROWS2 // BLK1


def _matvec_body(t_ref, m_ref, o_ref):
    o_ref[...] = jnp.dot(t_ref[...], m_ref[...],
                         preferred_element_type=jnp.float32)


def _stage1(table2, m):
    return pl.pallas_call(
        _matvec_body,
        grid=(GRID1,),
        in_specs=[
            pl.BlockSpec((BLK1, 128), lambda i: (i, 0)),
            pl.BlockSpec((128, 4), lambda i: (0, 0)),
        ],
        out_specs=pl.BlockSpec((BLK1, 4), lambda i: (i, 0)),
        out_shape=jax.ShapeDtypeStruct((ROWS2, 4), jnp.float32),
    )(table2, m)


NC = 2
NS = 16
NW = NC * NS
COLS_W = B_N // NW
CCH = 128
NCH = COLS_W // CCH
INV = 1.0 / (2 * L_N - 1)

_mesh = plsc.VectorSubcoreMesh(core_axis_name="c", subcore_axis_name="s")


NBLK = B_N // CCH
CHUNK = L_N * CCH


@functools.partial(
    pl.kernel,
    mesh=_mesh,
    out_type=jax.ShapeDtypeStruct((B_N,), jnp.float32),
    scratch_types=[
        pltpu.VMEM((CHUNK,), jnp.int32),
        pltpu.VMEM((CHUNK,), jnp.float32),
        pltpu.VMEM((CCH,), jnp.float32),
        pltpu.VMEM((16,), jnp.float32),
        pltpu.SemaphoreType.DMA,
    ],
)
def _pool_kernel(a_hbm, s_hbm, b16_hbm, out_hbm, idx_v, vals_v, outc_v, b_v,
                 sem):
    wid = lax.axis_index("s") * NC + lax.axis_index("c")
    pltpu.sync_copy(b16_hbm, b_v)
    bias = b_v[...]
    for ch in range(NCH):
        blk = wid * NCH + ch
        pltpu.sync_copy(a_hbm.at[blk], idx_v)
        pltpu.async_copy(s_hbm.at[idx_v], vals_v, sem).wait()
        for g in range(CCH // 16):
            off = g * 16

            def body(l, acc, off=off):
                return acc + vals_v[pl.ds(l * CCH + off, 16)]

            acc = lax.fori_loop(0, L_N, body, jnp.zeros((16,), jnp.float32))
            first = vals_v[pl.ds(off, 16)]
            last = vals_v[pl.ds((L_N - 1) * CCH + off, 16)]
            outc_v[pl.ds(off, 16)] = (3.0 * acc - first - last) * INV + bias
        pltpu.sync_copy(outc_v, out_hbm.at[pl.ds(blk * CCH, CCH)])


def kernel(x_in, table, W, b):
    table2 = table.reshape(ROWS2, 128)
    m = jnp.kron(jnp.eye(4, dtype=W.dtype), W)
    s = _stage1(table2, m).reshape(VOCAB_N)
    a = (x_in.astype(jnp.int32).T
         .reshape(L_N, NBLK, CCH).transpose(1, 0, 2).reshape(NBLK, CHUNK))
    b16 = jnp.broadcast_to(b, (16,))
    pooled = _pool_kernel(a, s, b16)
    return pooled.reshape(B_N, 1)


# speedup vs baseline: 13.7834x; 1.0185x over previous
import functools
import jax
import jax.numpy as jnp
from jax import lax
from jax.experimental import pallas as pl
from jax.experimental.pallas import tpu as pltpu
from jax.experimental.pallas import tpu_sc as plsc

VOCAB_N = 1000000
D_N = 32
B_N = 16384
L_N = 200

ROWS2 = VOCAB_N * D_N // 128
BLK1 = 2000
GRID1 = ROWS2 // BLK1


def _matvec_body(t_ref, m_ref, o_ref):
    o_ref[...] = jnp.dot(t_ref[...], m_ref[...],
                         preferred_element_type=jnp.float32)


def _stage1(table2, m):
    return pl.pallas_call(
        _matvec_body,
        grid=(GRID1,),
        in_specs=[
            pl.BlockSpec((BLK1, 128), lambda i: (i, 0)),
            pl.BlockSpec((128, 4), lambda i: (0, 0)),
        ],
        out_specs=pl.BlockSpec((BLK1, 4), lambda i: (i, 0)),
        out_shape=jax.ShapeDtypeStruct((ROWS2, 4), jnp.float32),
    )(table2, m)


NC = 2
NS = 16
NW = NC * NS
CCH = 128
NCH = (B_N // NW) // CCH
NBLK = B_N // CCH
CHUNK = L_N * CCH
INV = 1.0 / (2 * L_N - 1)


def _tr_body(x_ref, o_ref):
    o_ref[...] = x_ref[...].T[None]


def _transpose(x):
    return pl.pallas_call(
        _tr_body,
        grid=(NBLK,),
        in_specs=[pl.BlockSpec((CCH, L_N), lambda i: (i, 0))],
        out_specs=pl.BlockSpec((1, L_N, CCH), lambda i: (i, 0, 0)),
        out_shape=jax.ShapeDtypeStruct((NBLK, L_N, CCH), jnp.int32),
    )(x)


_mesh = plsc.VectorSubcoreMesh(core_axis_name="c", subcore_axis_name="s")


@functools.partial(
    pl.kernel,
    mesh=_mesh,
    out_type=jax.ShapeDtypeStruct((B_N,), jnp.float32),
    scratch_types=[
        pltpu.VMEM((CHUNK,), jnp.int32),
        pltpu.VMEM((CHUNK,), jnp.int32),
        pltpu.VMEM((CHUNK,), jnp.float32),
        pltpu.VMEM((CHUNK,), jnp.float32),
        pltpu.VMEM((CCH,), jnp.float32),
        pltpu.VMEM((16,), jnp.float32),
        pltpu.SemaphoreType.DMA,
        pltpu.SemaphoreType.DMA,
    ],
)
def _pool_kernel(a_hbm, s_hbm, b16_hbm, out_hbm, idx_a, idx_b, vals_a, vals_b,
                 outc_v, b_v, sem_a, sem_b):
    wid = lax.axis_index("s") * NC + lax.axis_index("c")
    pltpu.sync_copy(b16_hbm, b_v)
    bias = b_v[...]
    idx = [idx_a, idx_b]
    vals = [vals_a, vals_b]
    sems = [sem_a, sem_b]
    copies = [None, None]

    def start(ch):
        buf = ch % 2
        blk = wid * NCH + ch
        pltpu.sync_copy(a_hbm.at[pl.ds(blk * CHUNK, CHUNK)], idx[buf])
        copies[buf] = pltpu.async_copy(s_hbm.at[idx[buf]], vals[buf],
                                       sems[buf])

    start(0)
    for ch in range(NCH):
        buf = ch % 2
        if ch + 1 < NCH:
            start(ch + 1)
        copies[buf].wait()
        vv = vals[buf]
        for g in range(CCH // 16):
            off = g * 16

            def col_body(l, acc, off=off, vv=vv):
                return acc + vv[pl.ds(l * CCH + off, 16)]

            acc = lax.fori_loop(0, L_N, col_body, jnp.zeros((16,),
                                                            jnp.float32))
            first = vv[pl.ds(off, 16)]
            last = vv[pl.ds((L_N - 1) * CCH + off, 16)]
            outc_v[pl.ds(off, 16)] = (3.0 * acc - first - last) * INV + bias
        blk = wid * NCH + ch
        pltpu.sync_copy(outc_v, out_hbm.at[pl.ds(blk * CCH, CCH)])


def kernel(x_in, table, W, b):
    table2 = table.reshape(ROWS2, 128)
    m = jnp.kron(jnp.eye(4, dtype=W.dtype), W)
    s = _stage1(table2, m).reshape(VOCAB_N)
    a = _transpose(x_in.astype(jnp.int32)).reshape(NBLK * CHUNK)
    b16 = jnp.broadcast_to(b, (16,))
    pooled = _pool_kernel(a, s, b16)
    return pooled.reshape(B_N, 1)


# speedup vs baseline: 37.7367x; 2.7378x over previous
import functools
import jax
import jax.numpy as jnp
from jax import lax
from jax.experimental import pallas as pl
from jax.experimental.pallas import tpu as pltpu
from jax.experimental.pallas import tpu_sc as plsc

VOCAB_N = 1000000
D_N = 32
B_N = 16384
L_N = 200

VBLK = 8192
GRID1 = -(-VOCAB_N // VBLK)
SPAD = GRID1 * VBLK


def _matvec_body(t_ref, w_ref, o_ref):
    o_ref[...] = jnp.sum(t_ref[...] * w_ref[:, 0:1], axis=0)


def _stage1(table_t, w128):
    return pl.pallas_call(
        _matvec_body,
        grid=(GRID1,),
        in_specs=[
            pl.BlockSpec((D_N, VBLK), lambda i: (0, i)),
            pl.BlockSpec((D_N, 128), lambda i: (0, 0)),
        ],
        out_specs=pl.BlockSpec((VBLK,), lambda i: (i,)),
        out_shape=jax.ShapeDtypeStruct((SPAD,), jnp.float32),
    )(table_t, w128)


NC = 2
NS = 16
NW = NC * NS
CCH = 128
NCH = (B_N // NW) // CCH
NBLK = B_N // CCH
CHUNK = L_N * CCH
INV = 1.0 / (2 * L_N - 1)


def _chunk_body(x_ref, o_ref):
    o_ref[...] = x_ref[...][None]


def _chunk(x_t):
    return pl.pallas_call(
        _chunk_body,
        grid=(NBLK,),
        in_specs=[pl.BlockSpec((L_N, CCH), lambda i: (0, i))],
        out_specs=pl.BlockSpec((1, L_N, CCH), lambda i: (i, 0, 0)),
        out_shape=jax.ShapeDtypeStruct((NBLK, L_N, CCH), jnp.int32),
    )(x_t)


_mesh = plsc.VectorSubcoreMesh(core_axis_name="c", subcore_axis_name="s")


@functools.partial(
    pl.kernel,
    mesh=_mesh,
    out_type=jax.ShapeDtypeStruct((B_N,), jnp.float32),
    scratch_types=[
        pltpu.VMEM((CHUNK,), jnp.int32),
        pltpu.VMEM((CHUNK,), jnp.int32),
        pltpu.VMEM((CHUNK,), jnp.float32),
        pltpu.VMEM((CHUNK,), jnp.float32),
        pltpu.VMEM((CCH,), jnp.float32),
        pltpu.VMEM((16,), jnp.float32),
        pltpu.SemaphoreType.DMA,
        pltpu.SemaphoreType.DMA,
    ],
)
def _pool_kernel(a_hbm, s_hbm, b16_hbm, out_hbm, idx_a, idx_b, vals_a, vals_b,
                 outc_v, b_v, sem_a, sem_b):
    wid = lax.axis_index("s") * NC + lax.axis_index("c")
    pltpu.sync_copy(b16_hbm, b_v)
    bias = b_v[...]
    idx = [idx_a, idx_b]
    vals = [vals_a, vals_b]
    sems = [sem_a, sem_b]
    copies = [None, None]

    def start(ch):
        buf = ch % 2
        blk = wid * NCH + ch
        pltpu.sync_copy(a_hbm.at[pl.ds(blk * CHUNK, CHUNK)], idx[buf])
        copies[buf] = pltpu.async_copy(s_hbm.at[idx[buf]], vals[buf],
                                       sems[buf])

    start(0)
    for ch in range(NCH):
        buf = ch % 2
        if ch + 1 < NCH:
            start(ch + 1)
        copies[buf].wait()
        vv = vals[buf]
        for g in range(CCH // 16):
            off = g * 16

            def col_body(l, acc, off=off, vv=vv):
                return acc + vv[pl.ds(l * CCH + off, 16)]

            acc = lax.fori_loop(0, L_N, col_body, jnp.zeros((16,),
                                                            jnp.float32))
            first = vv[pl.ds(off, 16)]
            last = vv[pl.ds((L_N - 1) * CCH + off, 16)]
            outc_v[pl.ds(off, 16)] = (3.0 * acc - first - last) * INV + bias
        blk = wid * NCH + ch
        pltpu.sync_copy(outc_v, out_hbm.at[pl.ds(blk * CCH, CCH)])


def kernel(x_in, table, W, b):
    table_t = table.T
    w128 = jnp.broadcast_to(W, (D_N, 128))
    s = _stage1(table_t, w128)
    a = _chunk(x_in.astype(jnp.int32).T).reshape(NBLK * CHUNK)
    b16 = jnp.broadcast_to(b, (16,))
    pooled = _pool_kernel(a, s, b16)
    return pooled.reshape(B_N, 1)


# speedup vs baseline: 53.0166x; 1.4049x over previous
import functools
import jax
import jax.numpy as jnp
from jax import lax
from jax.experimental import pallas as pl
from jax.experimental.pallas import tpu as pltpu
from jax.experimental.pallas import tpu_sc as plsc

VOCAB_N = 1000000
D_N = 32
B_N = 16384
L_N = 200

VBLK = 8192
GRID1 = -(-VOCAB_N // VBLK)
SPAD = GRID1 * VBLK


def _matvec_body(t_ref, w_ref, o_ref):
    o_ref[...] = jnp.sum(t_ref[...] * w_ref[:, 0:1], axis=0)


def _stage1(table_t, w128):
    return pl.pallas_call(
        _matvec_body,
        grid=(GRID1,),
        in_specs=[
            pl.BlockSpec((D_N, VBLK), lambda i: (0, i)),
            pl.BlockSpec((D_N, 128), lambda i: (0, 0)),
        ],
        out_specs=pl.BlockSpec((VBLK,), lambda i: (i,)),
        out_shape=jax.ShapeDtypeStruct((SPAD,), jnp.float32),
    )(table_t, w128)


NC = 2
NS = 16
NW = NC * NS
CCH = 128
NCH = (B_N // NW) // CCH
NBLK = B_N // CCH
CHUNK = L_N * CCH
INV = 1.0 / (2 * L_N - 1)


def _chunk_body(x_ref, o_ref):
    o_ref[...] = x_ref[...][None]


def _chunk(x_t):
    return pl.pallas_call(
        _chunk_body,
        grid=(NBLK,),
        in_specs=[pl.BlockSpec((L_N, CCH), lambda i: (0, i))],
        out_specs=pl.BlockSpec((1, L_N, CCH), lambda i: (i, 0, 0)),
        out_shape=jax.ShapeDtypeStruct((NBLK, L_N, CCH), jnp.int32),
    )(x_t)


_mesh = plsc.VectorSubcoreMesh(core_axis_name="c", subcore_axis_name="s")


HALF = CHUNK // 2
LHALF = L_N // 2


@functools.partial(
    pl.kernel,
    mesh=_mesh,
    out_type=jax.ShapeDtypeStruct((B_N,), jnp.float32),
    scratch_types=[
        pltpu.VMEM((CHUNK,), jnp.int32),
        pltpu.VMEM((HALF,), jnp.float32),
        pltpu.VMEM((HALF,), jnp.float32),
        pltpu.VMEM((CCH,), jnp.float32),
        pltpu.VMEM((16,), jnp.float32),
        pltpu.VMEM_SHARED((SPAD,), jnp.float32),
        pltpu.SemaphoreType.DMA,
        pltpu.SemaphoreType.DMA,
        pltpu.SemaphoreType.DMA,
    ],
)
def _pool_kernel(a_hbm, s_hbm, b16_hbm, out_hbm, idx_v, vals_a, vals_b,
                 outc_v, b_v, s_sh, sem_a, sem_b, sem_i):
    wid = lax.axis_index("s") * NC + lax.axis_index("c")
    pltpu.sync_copy(b16_hbm, b_v)
    bias = b_v[...]

    @pl.when(lax.axis_index("s") == 0)
    def _():
        pltpu.sync_copy(s_hbm, s_sh)

    plsc.subcore_barrier()

    def start_gathers():
        ca = pltpu.async_copy(s_sh.at[idx_v.at[pl.ds(0, HALF)]], vals_a,
                              sem_a)
        cb = pltpu.async_copy(s_sh.at[idx_v.at[pl.ds(HALF, HALF)]], vals_b,
                              sem_b)
        return ca, cb

    pltpu.sync_copy(a_hbm.at[pl.ds(wid * NCH * CHUNK, CHUNK)], idx_v)
    ca, cb = start_gathers()
    for ch in range(NCH):
        ca.wait()
        cb.wait()
        ci = None
        if ch + 1 < NCH:
            blkn = wid * NCH + ch + 1
            ci = pltpu.async_copy(a_hbm.at[pl.ds(blkn * CHUNK, CHUNK)],
                                  idx_v, sem_i)
        for g in range(CCH // 16):
            off = g * 16

            def half_body(ll, acc, off=off):
                va = vals_a
                vb = vals_b
                for u in range(2):
                    acc = acc + va[pl.ds((ll * 2 + u) * CCH + off, 16)]
                for u in range(2):
                    acc = acc + vb[pl.ds((ll * 2 + u) * CCH + off, 16)]
                return acc

            acc = lax.fori_loop(0, LHALF // 2, half_body,
                                jnp.zeros((16,), jnp.float32))
            first = vals_a[pl.ds(off, 16)]
            last = vals_b[pl.ds((LHALF - 1) * CCH + off, 16)]
            outc_v[pl.ds(off, 16)] = (3.0 * acc - first - last) * INV + bias
        blk = wid * NCH + ch
        pltpu.sync_copy(outc_v, out_hbm.at[pl.ds(blk * CCH, CCH)])
        if ci is not None:
            ci.wait()
            ca, cb = start_gathers()


def kernel(x_in, table, W, b):
    table_t = table.T
    w128 = jnp.broadcast_to(W, (D_N, 128))
    s = _stage1(table_t, w128)
    a = _chunk(x_in.astype(jnp.int32).T).reshape(NBLK * CHUNK)
    b16 = jnp.broadcast_to(b, (16,))
    pooled = _pool_kernel(a, s, b16)
    return pooled.reshape(B_N, 1)


# speedup vs baseline: 90.5756x; 1.7084x over previous
import functools
import jax
import jax.numpy as jnp
from jax import lax
from jax.experimental import pallas as pl
from jax.experimental.pallas import tpu as pltpu
from jax.experimental.pallas import tpu_sc as plsc

VOCAB_N = 1000000
D_N = 32
B_N = 16384
L_N = 200

VBLK = 16384
GRID1 = -(-VOCAB_N // VBLK)
SPAD = GRID1 * VBLK


def _matvec_body(t_ref, w_ref, o_ref):
    o_ref[...] = jnp.sum(t_ref[...] * w_ref[:, 0:1], axis=0)


def _stage1(table_t, w128):
    return pl.pallas_call(
        _matvec_body,
        grid=(GRID1,),
        in_specs=[
            pl.BlockSpec((D_N, VBLK), lambda i: (0, i)),
            pl.BlockSpec((D_N, 128), lambda i: (0, 0)),
        ],
        out_specs=pl.BlockSpec((VBLK,), lambda i: (i,)),
        out_shape=jax.ShapeDtypeStruct((SPAD,), jnp.float32),
    )(table_t, w128)


NC = 2
NS = 16
NW = NC * NS
CCH = 128
NCH = (B_N // NW) // CCH
CHUNK = L_N * CCH
LA = 96
LB = L_N - LA
HALFA = LA * CCH
HALFB = LB * CCH
INV = 1.0 / (2 * L_N - 1)

_mesh = plsc.VectorSubcoreMesh(core_axis_name="c", subcore_axis_name="s")


@functools.partial(
    pl.kernel,
    mesh=_mesh,
    out_type=jax.ShapeDtypeStruct((B_N,), jnp.float32),
    scratch_types=[
        pltpu.VMEM((LA, CCH), jnp.int32),
        pltpu.VMEM((LB, CCH), jnp.int32),
        pltpu.VMEM((HALFA,), jnp.float32),
        pltpu.VMEM((HALFB,), jnp.float32),
        pltpu.VMEM((CCH,), jnp.float32),
        pltpu.VMEM((16,), jnp.float32),
        pltpu.VMEM_SHARED((SPAD,), jnp.float32),
        pltpu.SemaphoreType.DMA,
        pltpu.SemaphoreType.DMA,
        pltpu.SemaphoreType.DMA,
    ],
)
def _pool_kernel(xt_hbm, s_hbm, b16_hbm, out_hbm, idx_a, idx_b, vals_a,
                 vals_b, outc_v, b_v, s_sh, sem_a, sem_b, sem_i):
    wid = lax.axis_index("s") * NC + lax.axis_index("c")
    pltpu.sync_copy(b16_hbm, b_v)
    bias = b_v[...]

    @pl.when(lax.axis_index("s") == 0)
    def _():
        pltpu.sync_copy(s_hbm, s_sh)

    plsc.subcore_barrier()

    def load_idx(ch):
        col = (wid * NCH + ch) * CCH
        ia = pltpu.async_copy(
            xt_hbm.at[pl.ds(0, LA), pl.ds(col, CCH)], idx_a, sem_i)
        ib = pltpu.async_copy(
            xt_hbm.at[pl.ds(LA, LB), pl.ds(col, CCH)], idx_b, sem_i)
        return ia, ib

    def start_gathers():
        ca = pltpu.async_copy(s_sh.at[idx_a.reshape(1, HALFA).at[0]], vals_a, sem_a)
        cb = pltpu.async_copy(s_sh.at[idx_b.reshape(1, HALFB).at[0]], vals_b, sem_b)
        return ca, cb

    ia, ib = load_idx(0)
    ia.wait()
    ib.wait()
    ca, cb = start_gathers()
    for ch in range(NCH):
        ca.wait()
        cb.wait()
        ci = None
        if ch + 1 < NCH:
            ci = load_idx(ch + 1)
        for g in range(CCH // 16):
            off = g * 16

            def body_a(ll, acc, off=off):
                for u in range(4):
                    acc = acc + vals_a[pl.ds((ll * 4 + u) * CCH + off, 16)]
                return acc

            def body_b(ll, acc, off=off):
                for u in range(4):
                    acc = acc + vals_b[pl.ds((ll * 4 + u) * CCH + off, 16)]
                return acc

            acc = lax.fori_loop(0, LA // 4, body_a,
                                jnp.zeros((16,), jnp.float32))
            acc = lax.fori_loop(0, LB // 4, body_b, acc)
            first = vals_a[pl.ds(off, 16)]
            last = vals_b[pl.ds((LB - 1) * CCH + off, 16)]
            outc_v[pl.ds(off, 16)] = (3.0 * acc - first - last) * INV + bias
        blk = wid * NCH + ch
        pltpu.sync_copy(outc_v, out_hbm.at[pl.ds(blk * CCH, CCH)])
        if ci is not None:
            ci[0].wait()
            ci[1].wait()
            ca, cb = start_gathers()


def kernel(x_in, table, W, b):
    table_t = table.T
    w128 = jnp.broadcast_to(W, (D_N, 128))
    s = _stage1(table_t, w128)
    xt = x_in.astype(jnp.int32).T
    b16 = jnp.broadcast_to(b, (16,))
    pooled = _pool_kernel(xt, s, b16)
    return pooled.reshape(B_N, 1)


# speedup vs baseline: 105.9713x; 1.1700x over previous
import functools
import jax
import jax.numpy as jnp
from jax import lax
from jax.experimental import pallas as pl
from jax.experimental.pallas import tpu as pltpu
from jax.experimental.pallas import tpu_sc as plsc

VOCAB_N = 1000000
D_N = 32
B_N = 16384
L_N = 200

VBLK = 32768
GRID1 = -(-VOCAB_N // VBLK)
SPAD = GRID1 * VBLK


def _matvec_body(t_ref, w_ref, o_ref):
    o_ref[...] = jnp.sum(t_ref[...] * w_ref[:, 0:1], axis=0)


def _stage1(table_t, w128):
    return pl.pallas_call(
        _matvec_body,
        grid=(GRID1,),
        in_specs=[
            pl.BlockSpec((D_N, VBLK), lambda i: (0, i)),
            pl.BlockSpec((D_N, 128), lambda i: (0, 0)),
        ],
        out_specs=pl.BlockSpec((VBLK,), lambda i: (i,)),
        out_shape=jax.ShapeDtypeStruct((SPAD,), jnp.float32),
    )(table_t, w128)


NC = 2
NS = 16
NW = NC * NS
CCH = 128
NCH = (B_N // NW) // CCH
CHUNK = L_N * CCH
LA = 96
LB = L_N - LA
HALFA = LA * CCH
HALFB = LB * CCH
INV = 1.0 / (2 * L_N - 1)

_mesh = plsc.VectorSubcoreMesh(core_axis_name="c", subcore_axis_name="s")


@functools.partial(
    pl.kernel,
    mesh=_mesh,
    out_type=jax.ShapeDtypeStruct((B_N,), jnp.float32),
    scratch_types=[
        pltpu.VMEM((LA, CCH), jnp.int32),
        pltpu.VMEM((LB, CCH), jnp.int32),
        pltpu.VMEM((HALFA,), jnp.float32),
        pltpu.VMEM((HALFB,), jnp.float32),
        pltpu.VMEM((CCH,), jnp.float32),
        pltpu.VMEM((16,), jnp.float32),
        pltpu.VMEM_SHARED((SPAD,), jnp.float32),
        pltpu.SemaphoreType.DMA,
        pltpu.SemaphoreType.DMA,
        pltpu.SemaphoreType.DMA,
    ],
)
def _pool_kernel(xt_hbm, s_hbm, b16_hbm, out_hbm, idx_a, idx_b, vals_a,
                 vals_b, outc_v, b_v, s_sh, sem_a, sem_b, sem_i):
    wid = lax.axis_index("s") * NC + lax.axis_index("c")
    pltpu.sync_copy(b16_hbm, b_v)
    bias = b_v[...]

    def load_idx(ch):
        col = (wid * NCH + ch) * CCH
        ia = pltpu.async_copy(
            xt_hbm.at[pl.ds(0, LA), pl.ds(col, CCH)], idx_a, sem_i)
        ib = pltpu.async_copy(
            xt_hbm.at[pl.ds(LA, LB), pl.ds(col, CCH)], idx_b, sem_i)
        return ia, ib

    def start_gathers():
        ca = pltpu.async_copy(s_sh.at[idx_a.reshape(1, HALFA).at[0]], vals_a, sem_a)
        cb = pltpu.async_copy(s_sh.at[idx_b.reshape(1, HALFB).at[0]], vals_b, sem_b)
        return ca, cb

    ia, ib = load_idx(0)
    sid = lax.axis_index("s")
    spart = SPAD // NS
    pltpu.sync_copy(s_hbm.at[pl.ds(sid * spart, spart)],
                    s_sh.at[pl.ds(sid * spart, spart)])
    plsc.subcore_barrier()
    ia.wait()
    ib.wait()
    ca, cb = start_gathers()
    for ch in range(NCH):
        ca.wait()
        cb.wait()
        ci = None
        if ch + 1 < NCH:
            ci = load_idx(ch + 1)
        for g in range(CCH // 16):
            off = g * 16

            def body_a(ll, acc, off=off):
                for u in range(4):
                    acc = acc + vals_a[pl.ds((ll * 4 + u) * CCH + off, 16)]
                return acc

            def body_b(ll, acc, off=off):
                for u in range(4):
                    acc = acc + vals_b[pl.ds((ll * 4 + u) * CCH + off, 16)]
                return acc

            acc = lax.fori_loop(0, LA // 4, body_a,
                                jnp.zeros((16,), jnp.float32))
            acc = lax.fori_loop(0, LB // 4, body_b, acc)
            first = vals_a[pl.ds(off, 16)]
            last = vals_b[pl.ds((LB - 1) * CCH + off, 16)]
            outc_v[pl.ds(off, 16)] = (3.0 * acc - first - last) * INV + bias
        blk = wid * NCH + ch
        pltpu.sync_copy(outc_v, out_hbm.at[pl.ds(blk * CCH, CCH)])
        if ci is not None:
            ci[0].wait()
            ci[1].wait()
            ca, cb = start_gathers()


def kernel(x_in, table, W, b):
    table_t = table.T
    w128 = jnp.broadcast_to(W, (D_N, 128))
    s = _stage1(table_t, w128)
    xt = x_in.astype(jnp.int32).T
    b16 = jnp.broadcast_to(b, (16,))
    pooled = _pool_kernel(xt, s, b16)
    return pooled.reshape(B_N, 1)


# speedup vs baseline: 111.9882x; 1.0568x over previous
import functools
import jax
import jax.numpy as jnp
from jax import lax
from jax.experimental import pallas as pl
from jax.experimental.pallas import tpu as pltpu
from jax.experimental.pallas import tpu_sc as plsc

VOCAB_N = 1000000
D_N = 32
B_N = 16384
L_N = 200

VBLK = 65536
GRID1 = -(-VOCAB_N // VBLK)
SPAD = GRID1 * VBLK


def _matvec_body(t_ref, w_ref, o_ref):
    o_ref[...] = jnp.sum(t_ref[...] * w_ref[:, 0:1], axis=0)


def _stage1(table_t, w128):
    return pl.pallas_call(
        _matvec_body,
        grid=(GRID1,),
        in_specs=[
            pl.BlockSpec((D_N, VBLK), lambda i: (0, i)),
            pl.BlockSpec((D_N, 128), lambda i: (0, 0)),
        ],
        out_specs=pl.BlockSpec((VBLK,), lambda i: (i,)),
        out_shape=jax.ShapeDtypeStruct((SPAD,), jnp.float32),
    )(table_t, w128)


NC = 2
NS = 16
NW = NC * NS
CCH = 128
NCH = (B_N // NW) // CCH
CHUNK = L_N * CCH
LA = 96
LB = L_N - LA
HALFA = LA * CCH
HALFB = LB * CCH
INV = 1.0 / (2 * L_N - 1)

_mesh = plsc.VectorSubcoreMesh(core_axis_name="c", subcore_axis_name="s")


@functools.partial(
    pl.kernel,
    mesh=_mesh,
    out_type=jax.ShapeDtypeStruct((B_N,), jnp.float32),
    scratch_types=[
        pltpu.VMEM((LA, CCH), jnp.int32),
        pltpu.VMEM((LB, CCH), jnp.int32),
        pltpu.VMEM((HALFA,), jnp.float32),
        pltpu.VMEM((HALFB,), jnp.float32),
        pltpu.VMEM((CCH,), jnp.float32),
        pltpu.VMEM((16,), jnp.float32),
        pltpu.VMEM_SHARED((SPAD,), jnp.float32),
        pltpu.SemaphoreType.DMA,
        pltpu.SemaphoreType.DMA,
        pltpu.SemaphoreType.DMA,
    ],
)
def _pool_kernel(xt_hbm, s_hbm, b16_hbm, out_hbm, idx_a, idx_b, vals_a,
                 vals_b, outc_v, b_v, s_sh, sem_a, sem_b, sem_i):
    wid = lax.axis_index("s") * NC + lax.axis_index("c")
    pltpu.sync_copy(b16_hbm, b_v)
    bias = b_v[...]

    def load_idx(ch):
        col = (wid * NCH + ch) * CCH
        ia = pltpu.async_copy(
            xt_hbm.at[pl.ds(0, LA), pl.ds(col, CCH)], idx_a, sem_i)
        ib = pltpu.async_copy(
            xt_hbm.at[pl.ds(LA, LB), pl.ds(col, CCH)], idx_b, sem_i)
        return ia, ib

    def start_gathers():
        ca = pltpu.async_copy(s_sh.at[idx_a.reshape(1, HALFA).at[0]], vals_a, sem_a)
        cb = pltpu.async_copy(s_sh.at[idx_b.reshape(1, HALFB).at[0]], vals_b, sem_b)
        return ca, cb

    ia, ib = load_idx(0)
    sid = lax.axis_index("s")
    spart = SPAD // NS
    pltpu.sync_copy(s_hbm.at[pl.ds(sid * spart, spart)],
                    s_sh.at[pl.ds(sid * spart, spart)])
    plsc.subcore_barrier()
    ia.wait()
    ca = pltpu.async_copy(s_sh.at[idx_a.reshape(1, HALFA).at[0]], vals_a,
                          sem_a)
    ib.wait()
    cb = pltpu.async_copy(s_sh.at[idx_b.reshape(1, HALFB).at[0]], vals_b,
                          sem_b)
    for ch in range(NCH):
        ca.wait()
        cb.wait()
        ci = None
        if ch + 1 < NCH:
            ci = load_idx(ch + 1)
        for g in range(CCH // 16):
            off = g * 16

            def body_a(ll, acc, off=off):
                for u in range(4):
                    acc = acc + vals_a[pl.ds((ll * 4 + u) * CCH + off, 16)]
                return acc

            def body_b(ll, acc, off=off):
                for u in range(4):
                    acc = acc + vals_b[pl.ds((ll * 4 + u) * CCH + off, 16)]
                return acc

            acc = lax.fori_loop(0, LA // 4, body_a,
                                jnp.zeros((16,), jnp.float32))
            acc = lax.fori_loop(0, LB // 4, body_b, acc)
            first = vals_a[pl.ds(off, 16)]
            last = vals_b[pl.ds((LB - 1) * CCH + off, 16)]
            outc_v[pl.ds(off, 16)] = (3.0 * acc - first - last) * INV + bias
        blk = wid * NCH + ch
        pltpu.sync_copy(outc_v, out_hbm.at[pl.ds(blk * CCH, CCH)])
        if ci is not None:
            ci[0].wait()
            ci[1].wait()
            ca, cb = start_gathers()


def kernel(x_in, table, W, b):
    table_t = table.T
    w128 = jnp.broadcast_to(W, (D_N, 128))
    s = _stage1(table_t, w128)
    xt = x_in.astype(jnp.int32).T
    b16 = jnp.broadcast_to(b, (16,))
    pooled = _pool_kernel(xt, s, b16)
    return pooled.reshape(B_N, 1)
